# baseline (speedup 1.0000x reference)
import functools

import jax
import jax.numpy as jnp
from jax import lax
from jax.experimental import pallas as pl
from jax.experimental.pallas import tpu as pltpu
from jax.experimental.pallas import tpu_sc as plsc

N = 10000
E = 320000
D = 128
EPS = 1e-5

NC = 2
NS = 16
NW = NC * NS
EW = E // NW
C = 128
EWP = 10240
NCHUNK = EWP // C
NP = 10240
RPT = NP // NS
RZ = 128
NRW = RPT // RZ
WZ = 128
NWB = RPT // WZ


def _sc_agg_body(x_hbm, src_hbm, dst_hbm, out_hbm,
                 src_v, dst_v, rows_v, acc_sh, sem):
    c = lax.axis_index("c")
    s = lax.axis_index("s")
    wid = s * NC + c

    def zfill(i, carry):
        rows_v[i // 8, pl.ds((i % 8) * 16, 16)] = jnp.zeros((16,), jnp.float32)
        return carry

    lax.fori_loop(0, RZ * 8, zfill, 0)
    for k in range(NRW):
        pltpu.sync_copy(rows_v.at[pl.ds(0, RZ)],
                        acc_sh.at[pl.ds(s * RPT + k * RZ, RZ)])
    plsc.subcore_barrier()

    base = wid * EWP

    def body(t, carry):
        off = base + t * C
        pltpu.sync_copy(src_hbm.at[pl.ds(off, C)], src_v)
        pltpu.sync_copy(dst_hbm.at[pl.ds(off, C)], dst_v)
        pltpu.async_copy(x_hbm.at[src_v], rows_v, sem).wait()
        pltpu.sync_copy(rows_v, acc_sh.at[dst_v], add=True)
        return carry

    lax.fori_loop(0, NCHUNK, body, 0)
    plsc.subcore_barrier()

    for k in range(NWB):
        r = s * RPT + k * WZ
        pltpu.sync_copy(acc_sh.at[pl.ds(r, WZ)], out_hbm.at[pl.ds(c * NP + r, WZ)])


@functools.lru_cache(maxsize=None)
def _get_sc_agg():
    return pl.kernel(
        _sc_agg_body,
        mesh=plsc.VectorSubcoreMesh(core_axis_name="c", subcore_axis_name="s"),
        out_type=jax.ShapeDtypeStruct((2 * NP, D), jnp.float32),
        scratch_types=[
            pltpu.VMEM((C,), jnp.int32),
            pltpu.VMEM((C,), jnp.int32),
            pltpu.VMEM((C, D), jnp.float32),
            pltpu.VMEM_SHARED((NP, D), jnp.float32),
            pltpu.SemaphoreType.DMA,
        ],
    )


def _tc_layer_body(p_ref, W_ref, b_ref, g_ref, be_ref, o_ref):
    a = p_ref[:N, :] + p_ref[NP:NP + N, :]
    y = jnp.dot(a, W_ref[...], preferred_element_type=jnp.float32) + b_ref[...]
    mu = jnp.mean(y, axis=0, keepdims=True)
    d = y - mu
    var = jnp.mean(d * d, axis=0, keepdims=True)
    yn = d * lax.rsqrt(var + EPS) * g_ref[...] + be_ref[...]
    o_ref[...] = jnp.maximum(yn, 0.0)


def _tc_layer(parts, W, b, g, be):
    return pl.pallas_call(
        _tc_layer_body,
        out_shape=jax.ShapeDtypeStruct((N, D), jnp.float32),
    )(parts, W, b.reshape(1, D), g.reshape(1, D), be.reshape(1, D))


def kernel(x, edge_index, W1, b1, g1, be1, W2, b2, g2, be2):
    ei = edge_index.astype(jnp.int32)
    pad = ((0, 0), (0, EWP - EW))
    src = jnp.pad(ei[0].reshape(NW, EW), pad).reshape(-1)
    padv = jnp.broadcast_to(jnp.arange(EWP - EW, dtype=jnp.int32) + N,
                            (NW, EWP - EW))
    dst = jnp.concatenate([ei[1].reshape(NW, EW), padv], axis=1).reshape(-1)
    sc_agg = _get_sc_agg()
    p1 = sc_agg(x, src, dst)
    h1 = _tc_layer(p1, W1, b1, g1, be1)
    p2 = sc_agg(h1, src, dst)
    return _tc_layer(p2, W2, b2, g2, be2)


# speedup vs baseline: 2.0173x; 2.0173x over previous
import functools

import jax
import jax.numpy as jnp
from jax import lax
from jax.experimental import pallas as pl
from jax.experimental.pallas import tpu as pltpu
from jax.experimental.pallas import tpu_sc as plsc

N = 10000
E = 320000
D = 128
EPS = 1e-5

NC = 2
NS = 16
NW = NC * NS
EW = E // NW
C = 125
NCHUNK = EW // C
NP = 10240
RPT = NP // NS
RZ = 80
NRW = RPT // RZ
WZ = 128
NWB = RPT // WZ


def _sc_agg_body(x_hbm, src_hbm, dst_hbm, out_hbm,
                 src_v, dst_v, rows_v, acc_sh, sem):
    c = lax.axis_index("c")
    s = lax.axis_index("s")
    wid = s * NC + c

    def zfill(i, carry):
        rows_v[i // 8, pl.ds((i % 8) * 16, 16)] = jnp.zeros((16,), jnp.float32)
        return carry

    lax.fori_loop(0, RZ * 8, zfill, 0)
    for k in range(NRW):
        pltpu.sync_copy(rows_v.at[pl.ds(0, RZ)],
                        acc_sh.at[pl.ds(s * RPT + k * RZ, RZ)])
    plsc.subcore_barrier()

    cbase = wid * NCHUNK

    def body(t, carry):
        pltpu.sync_copy(src_hbm.at[cbase + t, 0], src_v)
        pltpu.sync_copy(dst_hbm.at[cbase + t, 0], dst_v)
        pltpu.async_copy(x_hbm.at[src_v], rows_v, sem).wait()
        pltpu.sync_copy(rows_v, acc_sh.at[dst_v], add=True)
        return carry

    lax.fori_loop(0, NCHUNK, body, 0)
    plsc.subcore_barrier()

    for k in range(NWB):
        r = s * RPT + k * WZ
        pltpu.sync_copy(acc_sh.at[pl.ds(r, WZ)], out_hbm.at[pl.ds(c * NP + r, WZ)])


@functools.lru_cache(maxsize=None)
def _get_sc_agg():
    return pl.kernel(
        _sc_agg_body,
        mesh=plsc.VectorSubcoreMesh(core_axis_name="c", subcore_axis_name="s"),
        out_type=jax.ShapeDtypeStruct((2 * NP, D), jnp.float32),
        scratch_types=[
            pltpu.VMEM((C,), jnp.int32),
            pltpu.VMEM((C,), jnp.int32),
            pltpu.VMEM((C, D), jnp.float32),
            pltpu.VMEM_SHARED((NP, D), jnp.float32),
            pltpu.SemaphoreType.DMA,
        ],
    )


def _tc_layer_body(p_ref, W_ref, b_ref, g_ref, be_ref, o_ref):
    a = p_ref[:N, :] + p_ref[NP:NP + N, :]
    y = jnp.dot(a, W_ref[...], preferred_element_type=jnp.float32) + b_ref[...]
    mu = jnp.mean(y, axis=0, keepdims=True)
    d = y - mu
    var = jnp.mean(d * d, axis=0, keepdims=True)
    yn = d * lax.rsqrt(var + EPS) * g_ref[...] + be_ref[...]
    o_ref[...] = jnp.maximum(yn, 0.0)


def _tc_layer(parts, W, b, g, be):
    return pl.pallas_call(
        _tc_layer_body,
        out_shape=jax.ShapeDtypeStruct((N, D), jnp.float32),
    )(parts, W, b.reshape(1, D), g.reshape(1, D), be.reshape(1, D))


def kernel(x, edge_index, W1, b1, g1, be1, W2, b2, g2, be2):
    ei = edge_index.astype(jnp.int32)
    src = ei[0].reshape(NW * NCHUNK, 1, C)
    dst = ei[1].reshape(NW * NCHUNK, 1, C)
    sc_agg = _get_sc_agg()
    p1 = sc_agg(x, src, dst)
    h1 = _tc_layer(p1, W1, b1, g1, be1)
    p2 = sc_agg(h1, src, dst)
    return _tc_layer(p2, W2, b2, g2, be2)


# speedup vs baseline: 4.1062x; 2.0354x over previous
import functools

import jax
import jax.numpy as jnp
from jax import lax
from jax.experimental import pallas as pl
from jax.experimental.pallas import tpu as pltpu
from jax.experimental.pallas import tpu_sc as plsc

N = 10000
E = 320000
D = 128
EPS = 1e-5

NC = 2
NS = 16
NW = NC * NS
EW = E // NW
C = 125
NCHUNK = EW // C
NP = 10240
RPT = NP // NS
RZ = 80
NRW = RPT // RZ
WZ = 128
NWB = RPT // WZ


def _sc_agg_body(x_hbm, e2_hbm, out_hbm,
                 idx0_v, idx1_v, idx2_v, idx3_v, rows0_v, rows1_v, acc_sh,
                 *sems):
    idx = ((idx0_v, idx1_v), (idx2_v, idx3_v))
    rows = (rows0_v, rows1_v)
    gsem = sems[0:2]
    ssem = sems[2:4]
    isem = ((sems[4], sems[5]), (sems[6], sems[7]))
    c = lax.axis_index("c")
    s = lax.axis_index("s")
    wid = s * NC + c
    cbase = wid * NCHUNK

    def zfill(i, carry):
        rows0_v[i // 8, pl.ds((i % 8) * 16, 16)] = jnp.zeros((16,), jnp.float32)
        return carry

    lax.fori_loop(0, RZ * 8, zfill, 0)
    for k in range(NRW):
        pltpu.sync_copy(rows0_v.at[pl.ds(0, RZ)],
                        acc_sh.at[pl.ds(s * RPT + k * RZ, RZ)])

    def idxload(t, S, k):
        pltpu.async_copy(e2_hbm.at[cbase + t], idx[S][k], isem[S][k])

    def iwait(S, k):
        pltpu.make_async_copy(e2_hbm.at[cbase], idx[S][k], isem[S][k]).wait()

    def gather(t, S, k):
        del t
        pltpu.async_copy(x_hbm.at[idx[S][k].at[0]], rows[S], gsem[S])

    def gwait(S):
        pltpu.make_async_copy(x_hbm.at[idx[S][0].at[0]], rows[S], gsem[S]).wait()

    def scatter(t, S, k):
        del t
        pltpu.async_copy(rows[S], acc_sh.at[idx[S][k].at[1]], ssem[S], add=True)

    def swait(S):
        pltpu.make_async_copy(rows[S], acc_sh.at[idx[S][0].at[1]], ssem[S]).wait()

    def step(t, u, a, b_, f):
        S = u % 2
        k = (u // 2) % 2
        if a:
            swait(S)
        if b_:
            idxload(t + 2, S, 1 - k)
        iwait(S, k)
        gather(t, S, k)
        if f:
            So = 1 - S
            ko = ((u - 1) // 2) % 2 if u > 0 else 1
            gwait(So)
            scatter(t - 1, So, ko)

    for j in range(4):
        idxload(j, j % 2, j // 2)
    plsc.subcore_barrier()

    step(0, 0, False, False, False)
    step(1, 1, False, False, True)
    step(2, 2, True, True, True)
    step(3, 3, True, True, True)

    def grp(g, carry):
        for u in range(4):
            t = 4 + g * 4 + u
            step(t, u, True, True, True)
        return carry

    NMAIN = ((NCHUNK - 4 - 2) // 4) * 4
    lax.fori_loop(0, NMAIN // 4, grp, 0)
    for t in range(4 + NMAIN, NCHUNK):
        step(t, t % 4, True, t + 2 < NCHUNK, True)
    SL = (NCHUNK - 1) % 2
    kL = ((NCHUNK - 1) // 2) % 2
    gwait(SL)
    scatter(NCHUNK - 1, SL, kL)
    swait(1 - SL)
    swait(SL)
    plsc.subcore_barrier()

    for k in range(NWB):
        r = s * RPT + k * WZ
        pltpu.sync_copy(acc_sh.at[pl.ds(r, WZ)], out_hbm.at[pl.ds(c * NP + r, WZ)])


@functools.lru_cache(maxsize=None)
def _get_sc_agg():
    return pl.kernel(
        _sc_agg_body,
        mesh=plsc.VectorSubcoreMesh(core_axis_name="c", subcore_axis_name="s"),
        out_type=jax.ShapeDtypeStruct((2 * NP, D), jnp.float32),
        scratch_types=[
            pltpu.VMEM((2, C), jnp.int32),
            pltpu.VMEM((2, C), jnp.int32),
            pltpu.VMEM((2, C), jnp.int32),
            pltpu.VMEM((2, C), jnp.int32),
            pltpu.VMEM((C, D), jnp.float32),
            pltpu.VMEM((C, D), jnp.float32),
            pltpu.VMEM_SHARED((NP, D), jnp.float32),
        ] + [pltpu.SemaphoreType.DMA] * 8,
    )


def _tc_layer_body(p_ref, W_ref, b_ref, g_ref, be_ref, o_ref):
    a = p_ref[:N, :] + p_ref[NP:NP + N, :]
    y = jnp.dot(a, W_ref[...], preferred_element_type=jnp.float32) + b_ref[...]
    mu = jnp.mean(y, axis=0, keepdims=True)
    d = y - mu
    var = jnp.mean(d * d, axis=0, keepdims=True)
    yn = d * lax.rsqrt(var + EPS) * g_ref[...] + be_ref[...]
    o_ref[...] = jnp.maximum(yn, 0.0)


def _tc_layer(parts, W, b, g, be):
    return pl.pallas_call(
        _tc_layer_body,
        out_shape=jax.ShapeDtypeStruct((N, D), jnp.float32),
    )(parts, W, b.reshape(1, D), g.reshape(1, D), be.reshape(1, D))


def kernel(x, edge_index, W1, b1, g1, be1, W2, b2, g2, be2):
    ei = edge_index.astype(jnp.int32)
    e2 = jnp.stack([ei[0].reshape(NW * NCHUNK, C),
                    ei[1].reshape(NW * NCHUNK, C)], axis=1)
    sc_agg = _get_sc_agg()
    p1 = sc_agg(x, e2)
    h1 = _tc_layer(p1, W1, b1, g1, be1)
    p2 = sc_agg(h1, e2)
    return _tc_layer(p2, W2, b2, g2, be2)


# speedup vs baseline: 4.3167x; 1.0513x over previous
import functools

import jax
import jax.numpy as jnp
from jax import lax
from jax.experimental import pallas as pl
from jax.experimental.pallas import tpu as pltpu
from jax.experimental.pallas import tpu_sc as plsc

N = 10000
E = 320000
D = 128
EPS = 1e-5

NC = 2
NS = 16
NW = NC * NS
EW = E // NW
C = 100
NCHUNK = EW // C
NP = 10240
RPT = NP // NS
RZ = 80
NRW = RPT // RZ
WZ = 128
NWB = RPT // WZ


def _sc_agg_body(x_hbm, e2_hbm, out_hbm,
                 idx0_v, idx1_v, idx2_v, idx3_v, idx4_v, idx5_v,
                 rows0_v, rows1_v, rows2_v, acc_sh, *sems):
    idx = ((idx0_v, idx1_v), (idx2_v, idx3_v), (idx4_v, idx5_v))
    rows = (rows0_v, rows1_v, rows2_v)
    gsem = sems[0:3]
    ssem = sems[3:6]
    isem = ((sems[6], sems[7]), (sems[8], sems[9]), (sems[10], sems[11]))
    c = lax.axis_index("c")
    s = lax.axis_index("s")
    wid = s * NC + c
    cbase = wid * NCHUNK

    def zfill(i, carry):
        rows0_v[i // 8, pl.ds((i % 8) * 16, 16)] = jnp.zeros((16,), jnp.float32)
        return carry

    lax.fori_loop(0, RZ * 8, zfill, 0)
    for k in range(NRW):
        pltpu.sync_copy(rows0_v.at[pl.ds(0, RZ)],
                        acc_sh.at[pl.ds(s * RPT + k * RZ, RZ)])

    def idxload(t, S, k):
        pltpu.async_copy(e2_hbm.at[cbase + t], idx[S][k], isem[S][k])

    def iwait(S, k):
        pltpu.make_async_copy(e2_hbm.at[cbase], idx[S][k], isem[S][k]).wait()

    def gather(t, S, k):
        del t
        pltpu.async_copy(x_hbm.at[idx[S][k].at[0]], rows[S], gsem[S])

    def gwait(S):
        pltpu.make_async_copy(x_hbm.at[idx[S][0].at[0]], rows[S], gsem[S]).wait()

    def scatter(t, S, k):
        del t
        pltpu.async_copy(rows[S], acc_sh.at[idx[S][k].at[1]], ssem[S], add=True)

    def swait(S):
        pltpu.make_async_copy(rows[S], acc_sh.at[idx[S][0].at[1]], ssem[S]).wait()

    def step(t, u, a, b_, f):
        S = u % 3
        k = (u // 3) % 2
        if a:
            swait(S)
        if b_:
            ub = (u + 2) % 6
            idxload(t + 2, ub % 3, (ub // 3) % 2)
        iwait(S, k)
        gather(t, S, k)
        if f:
            uo = (u + 5) % 6
            gwait(uo % 3)
            scatter(t - 1, uo % 3, (uo // 3) % 2)

    for j in range(6):
        idxload(j, j % 3, j // 3)
    plsc.subcore_barrier()

    step(0, 0, False, False, False)
    step(1, 1, False, False, True)
    step(2, 2, False, False, True)
    step(3, 3, True, False, True)
    step(4, 4, True, True, True)
    step(5, 5, True, True, True)

    def grp(g, carry):
        for u in range(6):
            t = 6 + g * 6 + u
            step(t, u, True, True, True)
        return carry

    NMAIN = ((NCHUNK - 6 - 4) // 6) * 6
    lax.fori_loop(0, NMAIN // 6, grp, 0)
    for t in range(6 + NMAIN, NCHUNK):
        step(t, t % 6, True, t + 2 < NCHUNK, True)
    uL = (NCHUNK - 1) % 6
    gwait(uL % 3)
    scatter(NCHUNK - 1, uL % 3, (uL // 3) % 2)
    for S in range(3):
        swait(S)
    plsc.subcore_barrier()

    for k in range(NWB):
        r = s * RPT + k * WZ
        pltpu.sync_copy(acc_sh.at[pl.ds(r, WZ)], out_hbm.at[pl.ds(c * NP + r, WZ)])


@functools.lru_cache(maxsize=None)
def _get_sc_agg():
    return pl.kernel(
        _sc_agg_body,
        mesh=plsc.VectorSubcoreMesh(core_axis_name="c", subcore_axis_name="s"),
        out_type=jax.ShapeDtypeStruct((2 * NP, D), jnp.float32),
        scratch_types=[
            pltpu.VMEM((2, C), jnp.int32),
            pltpu.VMEM((2, C), jnp.int32),
            pltpu.VMEM((2, C), jnp.int32),
            pltpu.VMEM((2, C), jnp.int32),
            pltpu.VMEM((2, C), jnp.int32),
            pltpu.VMEM((2, C), jnp.int32),
            pltpu.VMEM((C, D), jnp.float32),
            pltpu.VMEM((C, D), jnp.float32),
            pltpu.VMEM((C, D), jnp.float32),
            pltpu.VMEM_SHARED((NP, D), jnp.float32),
        ] + [pltpu.SemaphoreType.DMA] * 12,
    )


def _tc_layer_body(p_ref, W_ref, b_ref, g_ref, be_ref, o_ref):
    a = p_ref[:N, :] + p_ref[NP:NP + N, :]
    y = jnp.dot(a, W_ref[...], preferred_element_type=jnp.float32) + b_ref[...]
    mu = jnp.mean(y, axis=0, keepdims=True)
    d = y - mu
    var = jnp.mean(d * d, axis=0, keepdims=True)
    yn = d * lax.rsqrt(var + EPS) * g_ref[...] + be_ref[...]
    o_ref[...] = jnp.maximum(yn, 0.0)


def _tc_layer(parts, W, b, g, be):
    return pl.pallas_call(
        _tc_layer_body,
        out_shape=jax.ShapeDtypeStruct((N, D), jnp.float32),
    )(parts, W, b.reshape(1, D), g.reshape(1, D), be.reshape(1, D))


def kernel(x, edge_index, W1, b1, g1, be1, W2, b2, g2, be2):
    ei = edge_index.astype(jnp.int32)
    e2 = jnp.stack([ei[0].reshape(NW * NCHUNK, C),
                    ei[1].reshape(NW * NCHUNK, C)], axis=1)
    sc_agg = _get_sc_agg()
    p1 = sc_agg(x, e2)
    h1 = _tc_layer(p1, W1, b1, g1, be1)
    p2 = sc_agg(h1, e2)
    return _tc_layer(p2, W2, b2, g2, be2)
